# Initial kernel scaffold; baseline (speedup 1.0000x reference)
import jax
import jax.numpy as jnp
from jax.experimental import pallas as pl


def kernel(x):
    raise NotImplementedError("write your pallas kernel here")


# speedup vs baseline: 2.6585x; 2.6585x over previous
import functools

import jax
import jax.numpy as jnp
from jax import lax
from jax.experimental import pallas as pl
from jax.experimental.pallas import tpu as pltpu
from jax.experimental.pallas import tpu_sc as plsc

_K = 64
_ROWS = 128
_COLS = 32768
_LANES = 16
_NVEC = _COLS // _LANES
_NCORES = 2
_NSUBCORES = 16
_NWORKERS = _NCORES * _NSUBCORES
_ROWS_PER_W = _ROWS // _NWORKERS

_mesh = plsc.VectorSubcoreMesh(
    core_axis_name="c", subcore_axis_name="s",
    num_cores=_NCORES, num_subcores=_NSUBCORES)


def _key16(iv):
  return jnp.where(iv < 0, iv ^ jnp.int32(0x7FFFFFFF), iv)


def _scan_hist(h, ngroups, k_rem):
  def body(st):
    g, run, _ = st
    gv = h[pl.ds(g * _LANES, _LANES)]
    s = jnp.sum(gv)
    done = run + s >= k_rem
    g_next = jnp.where(done, g, g - 1)
    run_next = jnp.where(done, run, run + s)
    return (g_next, run_next, done)

  def cond(st):
    return jnp.logical_not(st[2])

  g, run, _ = lax.while_loop(
      cond, body,
      (jnp.int32(ngroups - 1), jnp.int32(0), jnp.bool_(False)))

  gv = h[pl.ds(g * _LANES, _LANES)]
  rv = lax.rev(gv, (0,))
  cs = plsc.cumsum(rv)
  need = k_rem - run
  j = jnp.sum((cs < need).astype(jnp.int32))
  iota = lax.iota(jnp.int32, _LANES)
  above_in_grp = jnp.sum(jnp.where(iota < j, rv, 0))
  bin_idx = g * _LANES + (_LANES - 1) - j
  return bin_idx, run + above_in_grp


@functools.partial(
    pl.kernel,
    out_type=jax.ShapeDtypeStruct((_ROWS, _COLS), jnp.int32),
    mesh=_mesh,
    compiler_params=pltpu.CompilerParams(needs_layout_passes=False),
    scratch_types=[
        pltpu.VMEM((_COLS,), jnp.int32),
        pltpu.VMEM((_COLS,), jnp.int32),
        pltpu.VMEM((2048,), jnp.int32),
        pltpu.VMEM((2048,), jnp.int32),
        pltpu.VMEM((1024,), jnp.int32),
    ],
)
def _topk_mask(x_hbm, out_hbm, xb, ob, h1, h2, h3):
  wid = lax.axis_index("s") * _NCORES + lax.axis_index("c")
  ones = jnp.ones((_LANES,), jnp.int32)
  zeros = jnp.zeros((_LANES,), jnp.int32)

  def do_row(r, _):
    row = wid * _ROWS_PER_W + r
    pltpu.sync_copy(x_hbm.at[row], xb)

    def zero128(i, _):
      h1[pl.ds(i * _LANES, _LANES)] = zeros
      h2[pl.ds(i * _LANES, _LANES)] = zeros
      return 0
    lax.fori_loop(0, 128, zero128, 0)

    def zero64(i, _):
      h3[pl.ds(i * _LANES, _LANES)] = zeros
      return 0
    lax.fori_loop(0, 64, zero64, 0)

    def pass_a(i, _):
      key = _key16(xb[pl.ds(i * _LANES, _LANES)])
      b1 = (key >> 21) + 1024
      plsc.addupdate_scatter(h1, [b1], ones)
      return 0
    lax.fori_loop(0, _NVEC, pass_a, 0)

    b1_bin, above1 = _scan_hist(h1, 128, jnp.int32(_K))
    p1 = b1_bin - 1024
    r1 = _K - above1

    def pass_b(i, _):
      key = _key16(xb[pl.ds(i * _LANES, _LANES)])
      m = (key >> 21) == p1
      b2 = (key >> 10) & 0x7FF
      plsc.addupdate_scatter(h2, [b2], ones, mask=m)
      return 0
    lax.fori_loop(0, _NVEC, pass_b, 0)

    b2_bin, above2 = _scan_hist(h2, 128, r1)
    p2 = (p1 << 11) | b2_bin
    r2 = r1 - above2

    def pass_c(i, _):
      key = _key16(xb[pl.ds(i * _LANES, _LANES)])
      m = (key >> 10) == p2
      b3 = key & 0x3FF
      plsc.addupdate_scatter(h3, [b3], ones, mask=m)
      return 0
    lax.fori_loop(0, _NVEC, pass_c, 0)

    b3_bin, above3 = _scan_hist(h3, 64, r2)
    t_key = (p2 << 10) | b3_bin
    r3 = r2 - above3

    def pass_d(i, run_eq):
      iv = xb[pl.ds(i * _LANES, _LANES)]
      key = _key16(iv)
      gt = key > t_key
      eq = key == t_key
      eqi = eq.astype(jnp.int32)
      pc = plsc.cumsum(eqi)
      keep_eq = jnp.logical_and(eq, (run_eq + pc) <= r3)
      keep = jnp.logical_and(jnp.logical_or(gt, keep_eq), iv > 0)
      ob[pl.ds(i * _LANES, _LANES)] = jnp.where(keep, iv, 0)
      return run_eq + jnp.sum(eqi)
    lax.fori_loop(0, _NVEC, pass_d, jnp.int32(0))

    pltpu.sync_copy(ob, out_hbm.at[row])
    return 0

  lax.fori_loop(0, _ROWS_PER_W, do_row, 0)


def kernel(x):
  xi = lax.bitcast_convert_type(x, jnp.int32)
  out_i = _topk_mask(xi)
  return lax.bitcast_convert_type(out_i, jnp.float32)


# speedup vs baseline: 8.4663x; 3.1846x over previous
import functools

import jax
import jax.numpy as jnp
from jax import lax
from jax.experimental import pallas as pl
from jax.experimental.pallas import tpu as pltpu
from jax.experimental.pallas import tpu_sc as plsc

_K = 64
_ROWS = 128
_COLS = 32768
_LANES = 16
_NVEC = _COLS // _LANES
_NCORES = 2
_NSUBCORES = 16
_NWORKERS = _NCORES * _NSUBCORES
_ROWS_PER_W = _ROWS // _NWORKERS

_mesh = plsc.VectorSubcoreMesh(
    core_axis_name="c", subcore_axis_name="s",
    num_cores=_NCORES, num_subcores=_NSUBCORES)


def _key16(iv):
  return jnp.where(iv < 0, iv ^ jnp.int32(0x7FFFFFFF), iv)


def _scan_hist(h, ngroups, k_rem):
  def body(st):
    g, run, _ = st
    gv = h[pl.ds(g * _LANES, _LANES)]
    s = jnp.sum(gv)
    done = run + s >= k_rem
    g_next = jnp.where(done, g, g - 1)
    run_next = jnp.where(done, run, run + s)
    return (g_next, run_next, done)

  def cond(st):
    return jnp.logical_not(st[2])

  g, run, _ = lax.while_loop(
      cond, body,
      (jnp.int32(ngroups - 1), jnp.int32(0), jnp.bool_(False)))

  gv = h[pl.ds(g * _LANES, _LANES)]
  rv = lax.rev(gv, (0,))
  cs = plsc.cumsum(rv)
  need = k_rem - run
  j = jnp.sum((cs < need).astype(jnp.int32))
  iota = lax.iota(jnp.int32, _LANES)
  above_in_grp = jnp.sum(jnp.where(iota < j, rv, 0))
  cnt_eq = jnp.sum(jnp.where(iota == j, rv, 0))
  bin_idx = g * _LANES + (_LANES - 1) - j
  return bin_idx, run + above_in_grp, cnt_eq


@functools.partial(
    pl.kernel,
    out_type=jax.ShapeDtypeStruct((_ROWS, _COLS), jnp.int32),
    mesh=_mesh,
    compiler_params=pltpu.CompilerParams(needs_layout_passes=False),
    scratch_types=[
        pltpu.VMEM((_COLS,), jnp.int32),
        pltpu.VMEM((_COLS,), jnp.int32),
        pltpu.VMEM((2048,), jnp.int32),
        pltpu.VMEM((2048,), jnp.int32),
        pltpu.VMEM((1024,), jnp.int32),
    ],
)
def _topk_mask(x_hbm, out_hbm, xb, ob, h1, h2, h3):
  wid = lax.axis_index("s") * _NCORES + lax.axis_index("c")
  ones = jnp.ones((_LANES,), jnp.int32)
  zeros = jnp.zeros((_LANES,), jnp.int32)

  def do_row(r, _):
    row = wid * _ROWS_PER_W + r
    pltpu.sync_copy(x_hbm.at[row], xb)

    @plsc.parallel_loop(0, 2048, step=_LANES, unroll=8)
    def _(i):
      h1[pl.ds(i, _LANES)] = zeros
      h2[pl.ds(i, _LANES)] = zeros

    @plsc.parallel_loop(0, 1024, step=_LANES, unroll=8)
    def _(i):
      h3[pl.ds(i, _LANES)] = zeros

    @plsc.parallel_loop(0, _COLS, step=_LANES, unroll=8)
    def _(i):
      key = _key16(xb[pl.ds(i, _LANES)])
      b1 = (key >> 21) + 1024
      plsc.addupdate_scatter(h1, [b1], ones)

    b1_bin, above1, _e1 = _scan_hist(h1, 128, jnp.int32(_K))
    p1 = b1_bin - 1024
    r1 = _K - above1

    @plsc.parallel_loop(0, _COLS, step=_LANES, unroll=8)
    def _(i):
      key = _key16(xb[pl.ds(i, _LANES)])
      m = (key >> 21) == p1
      b2 = (key >> 10) & 0x7FF
      plsc.addupdate_scatter(h2, [b2], ones, mask=m)

    b2_bin, above2, _e2 = _scan_hist(h2, 128, r1)
    p2 = (p1 << 11) | b2_bin
    r2 = r1 - above2

    @plsc.parallel_loop(0, _COLS, step=_LANES, unroll=8)
    def _(i):
      key = _key16(xb[pl.ds(i, _LANES)])
      m = (key >> 10) == p2
      b3 = key & 0x3FF
      plsc.addupdate_scatter(h3, [b3], ones, mask=m)

    b3_bin, above3, cnt_eq = _scan_hist(h3, 64, r2)
    t_key = (p2 << 10) | b3_bin
    excess = cnt_eq - (r2 - above3)

    @plsc.parallel_loop(0, _COLS, step=_LANES, unroll=8)
    def _(i):
      iv = xb[pl.ds(i, _LANES)]
      key = _key16(iv)
      keep = jnp.logical_and(key >= t_key, iv > 0)
      ob[pl.ds(i, _LANES)] = jnp.where(keep, iv, 0)

    def fix_body(st):
      i, ex = st
      iv = xb[pl.ds(i, _LANES)]
      eqi = (_key16(iv) == t_key).astype(jnp.int32)
      c = jnp.sum(eqi)
      rc = lax.rev(plsc.cumsum(lax.rev(eqi, (0,))), (0,))
      clear = jnp.logical_and(eqi > 0, rc <= ex)
      ov = ob[pl.ds(i, _LANES)]
      ob[pl.ds(i, _LANES)] = jnp.where(clear, 0, ov)
      return (i - _LANES, jnp.maximum(ex - c, 0))

    def fix_cond(st):
      i, ex = st
      return jnp.logical_and(ex > 0, i >= 0)

    lax.while_loop(fix_cond, fix_body,
                   (jnp.int32(_COLS - _LANES), excess))

    pltpu.sync_copy(ob, out_hbm.at[row])
    return 0

  lax.fori_loop(0, _ROWS_PER_W, do_row, 0)


def kernel(x):
  xi = lax.bitcast_convert_type(x, jnp.int32)
  out_i = _topk_mask(xi)
  return lax.bitcast_convert_type(out_i, jnp.float32)
